# Initial kernel scaffold; baseline (speedup 1.0000x reference)
import jax
import jax.numpy as jnp
from jax.experimental import pallas as pl


def kernel(pred_logits, pred_boxes, tgt_labels, tgt_boxes):
    raise NotImplementedError("write your pallas kernel here")


# speedup vs baseline: 1.1414x; 1.1414x over previous
import functools

import jax
import jax.numpy as jnp
from jax.experimental import pallas as pl
from jax.experimental.pallas import tpu as pltpu

B, Q, T, C = 16, 300, 20, 91
NCLS = C + 1

W_SUM = B * (0.1 * (Q - T) + 1.0 * T)
NUM_BOXES = float(B * T)


def _criterion_kernel(logits_ref, pbT_ref, lab_ref, tb_ref, out_ref):
    b = pl.program_id(0)

    logits = logits_ref[0]
    pbT = pbT_ref[0]
    lab = lab_ref[0]
    tb = tb_ref[0]

    rowmax = jnp.max(logits, axis=1, keepdims=True)
    ex = jnp.exp(logits - rowmax)
    se = jnp.sum(ex, axis=1, keepdims=True)
    prob = ex / se
    lse = jnp.log(se) + rowmax

    cls_iota = jax.lax.broadcasted_iota(jnp.int32, (NCLS, 1), 0)
    onehotL2 = (cls_iota == lab).astype(jnp.float32)
    cost_class = -jax.lax.dot_general(
        onehotL2, prob, (((0,), (1,)), ((), ())),
        preferred_element_type=jnp.float32)

    pcx, pcy, pw, ph = pbT[0:1], pbT[1:2], pbT[2:3], pbT[3:4]
    tcx, tcy, tw, th = tb[:, 0:1], tb[:, 1:2], tb[:, 2:3], tb[:, 3:4]
    cost_bbox = (jnp.abs(pcx - tcx) + jnp.abs(pcy - tcy)
                 + jnp.abs(pw - tw) + jnp.abs(ph - th))

    px1, py1 = pcx - 0.5 * pw, pcy - 0.5 * ph
    px2, py2 = pcx + 0.5 * pw, pcy + 0.5 * ph
    tx1, ty1 = tcx - 0.5 * tw, tcy - 0.5 * th
    tx2, ty2 = tcx + 0.5 * tw, tcy + 0.5 * th
    area_p = (px2 - px1) * (py2 - py1)
    area_t = (tx2 - tx1) * (ty2 - ty1)
    iw = jnp.clip(jnp.minimum(px2, tx2) - jnp.maximum(px1, tx1), 0.0)
    ih = jnp.clip(jnp.minimum(py2, ty2) - jnp.maximum(py1, ty1), 0.0)
    inter = iw * ih
    union = area_p + area_t - inter
    iou = inter / (union + 1e-8)
    hw = jnp.clip(jnp.maximum(px2, tx2) - jnp.minimum(px1, tx1), 0.0)
    hh = jnp.clip(jnp.maximum(py2, ty2) - jnp.minimum(py1, ty1), 0.0)
    hull = hw * hh
    giou = iou - (hull - union) / (hull + 1e-8)

    cost = cost_class + 5.0 * cost_bbox + (-2.0) * giou

    q_iota = jax.lax.broadcasted_iota(jnp.int32, (1, Q), 1)
    t_iota = jax.lax.broadcasted_iota(jnp.int32, (T, 1), 0)

    def body(t, carry):
        used, S = carry
        rowsel = (t_iota == t).astype(jnp.float32)
        c_t = jnp.sum(rowsel * cost, axis=0, keepdims=True)
        cm = jnp.where(used > 0.0, jnp.inf, c_t)
        m = jnp.min(cm)
        idx = jnp.min(jnp.where(cm == m, q_iota, Q))
        colsel = (q_iota == idx).astype(jnp.float32)
        return used + colsel, S + rowsel * colsel

    used0 = jnp.zeros((1, Q), jnp.float32)
    S0 = jnp.zeros((T, Q), jnp.float32)
    _, S = jax.lax.fori_loop(0, T, body, (used0, S0))

    col_eos = logits[:, C:C + 1]
    total_eos = jnp.sum(lse - col_eos)
    SL = jax.lax.dot_general(S, logits, (((1,), (0,)), ((), ())),
                             preferred_element_type=jnp.float32)
    slse = jax.lax.dot_general(S, lse, (((1,), (0,)), ((), ())),
                               preferred_element_type=jnp.float32)
    matched_lse = jnp.sum(slse)
    matched_eos = jnp.sum(SL[:, C:C + 1])
    M = jax.lax.dot_general(SL, onehotL2, (((1,), (0,)), ((), ())),
                            preferred_element_type=jnp.float32)
    tt_r = jax.lax.broadcasted_iota(jnp.int32, (T, T), 0)
    tt_c = jax.lax.broadcasted_iota(jnp.int32, (T, T), 1)
    matched_logit = jnp.sum(jnp.where(tt_r == tt_c, M, 0.0))
    ce_part = (0.1 * (total_eos - matched_lse + matched_eos)
               + (matched_lse - matched_logit))

    mb = jax.lax.dot_general(S, pbT, (((1,), (1,)), ((), ())),
                             preferred_element_type=jnp.float32)
    bbox_part = jnp.sum(jnp.abs(mb - tb))

    mcx, mcy, mw, mh = mb[:, 0:1], mb[:, 1:2], mb[:, 2:3], mb[:, 3:4]
    mx1, my1 = mcx - 0.5 * mw, mcy - 0.5 * mh
    mx2, my2 = mcx + 0.5 * mw, mcy + 0.5 * mh
    area_m = (mx2 - mx1) * (my2 - my1)
    eiw = jnp.clip(jnp.minimum(mx2, tx2) - jnp.maximum(mx1, tx1), 0.0)
    eih = jnp.clip(jnp.minimum(my2, ty2) - jnp.maximum(my1, ty1), 0.0)
    einter = eiw * eih
    eunion = area_m + area_t - einter
    eiou = einter / (eunion + 1e-8)
    ehw = jnp.clip(jnp.maximum(mx2, tx2) - jnp.minimum(mx1, tx1), 0.0)
    ehh = jnp.clip(jnp.maximum(my2, ty2) - jnp.minimum(my1, ty1), 0.0)
    ehull = ehw * ehh
    eg = eiou - (ehull - eunion) / (ehull + 1e-8)
    giou_part = jnp.sum(1.0 - eg)

    r8 = jax.lax.broadcasted_iota(jnp.int32, (8, 128), 0)
    c128 = jax.lax.broadcasted_iota(jnp.int32, (8, 128), 1)
    vals = (jnp.where((r8 == 0) & (c128 == 0), ce_part * (1.0 / W_SUM), 0.0)
            + jnp.where((r8 == 0) & (c128 == 1), bbox_part * (1.0 / NUM_BOXES), 0.0)
            + jnp.where((r8 == 0) & (c128 == 2), giou_part * (1.0 / NUM_BOXES), 0.0))

    @pl.when(b == 0)
    def _init():
        out_ref[...] = vals

    @pl.when(b > 0)
    def _acc():
        out_ref[...] += vals


@functools.partial(jax.jit, static_argnames=())
def kernel(pred_logits, pred_boxes, tgt_labels, tgt_boxes):
    pbT = jnp.transpose(pred_boxes, (0, 2, 1))
    lab3 = tgt_labels.astype(jnp.int32).reshape(B, 1, T)

    out = pl.pallas_call(
        _criterion_kernel,
        grid=(B,),
        in_specs=[
            pl.BlockSpec((1, Q, NCLS), lambda b: (b, 0, 0)),
            pl.BlockSpec((1, 4, Q), lambda b: (b, 0, 0)),
            pl.BlockSpec((1, 1, T), lambda b: (b, 0, 0)),
            pl.BlockSpec((1, T, 4), lambda b: (b, 0, 0)),
        ],
        out_specs=pl.BlockSpec((8, 128), lambda b: (0, 0)),
        out_shape=jax.ShapeDtypeStruct((8, 128), jnp.float32),
        compiler_params=pltpu.CompilerParams(
            dimension_semantics=("arbitrary",)),
    )(pred_logits, pbT, lab3, tgt_boxes)

    return (out[0, 0], out[0, 1], out[0, 2])


# speedup vs baseline: 3.2743x; 2.8685x over previous
import functools

import jax
import jax.numpy as jnp
from jax import lax
from jax.experimental import pallas as pl
from jax.experimental.pallas import tpu as pltpu
from jax.experimental.pallas import tpu_sc as plsc

B, Q, T, C = 16, 300, 20, 91
NCLS = C + 1
CP = 96
QP = 304
TP = 32
NQB = QP // 16
W_SUM = B * (0.1 * (Q - T) + 1.0 * T)
NUM_BOXES = float(B * T)
BIG = 1e30
LN2 = 0.6931471805599453


def _f(x):
    return jnp.full((16,), x, jnp.float32)


def _i(x):
    return jnp.full((16,), x, jnp.int32)


def _bcast(vec, j):
    return jnp.take_along_axis(vec, _i(j), axis=0,
                               mode=lax.GatherScatterMode.PROMISE_IN_BOUNDS)


def _log16(s):
    bits = lax.bitcast_convert_type(s, jnp.int32)
    e = ((bits >> 23) - 127).astype(jnp.float32)
    m = lax.bitcast_convert_type(
        (bits & 0x007FFFFF) | 0x3F800000, jnp.float32)
    big = m > 1.4142135
    m = jnp.where(big, 0.5 * m, m)
    e = jnp.where(big, e + 1.0, e)
    z = (m - 1.0) / (m + 1.0)
    z2 = z * z
    p = 2.0 * z * (1.0 + z2 * (1.0 / 3.0 + z2 * (0.2 + z2 * (1.0 / 7.0))))
    return e * LN2 + p


def _sc_body(logitsT_hbm, pbT_hbm, tbT_hbm, lab_hbm, out_hbm,
             logitsT, pbT, tbT, lab, lse, cost, res):
    wid = lax.axis_index("s") * 2 + lax.axis_index("c")

    @pl.when(wid < B)
    def _run():
        b = wid
        pltpu.sync_copy(logitsT_hbm.at[b], logitsT)
        pltpu.sync_copy(pbT_hbm.at[b], pbT)
        pltpu.sync_copy(tbT_hbm.at[b], tbT)
        pltpu.sync_copy(lab_hbm.at[b], lab)

        lane = jnp.arange(16, dtype=jnp.int32)

        def lse_block(qb, carry):
            sl = pl.ds(qb * 16, 16)
            m = logitsT[0, sl]
            for c in range(1, CP):
                m = jnp.maximum(m, logitsT[c, sl])
            s = _f(0.0)
            for c in range(CP):
                s = s + jnp.exp(logitsT[c, sl] - m)
            lse[sl] = _log16(s) + m
            return carry

        lax.fori_loop(0, NQB, lse_block, 0)

        labA, labB = lab[0:16], lab[16:32]
        tbxA, tbxB = tbT[0, 0:16], tbT[0, 16:32]
        tbyA, tbyB = tbT[1, 0:16], tbT[1, 16:32]
        tbwA, tbwB = tbT[2, 0:16], tbT[2, 16:32]
        tbhA, tbhB = tbT[3, 0:16], tbT[3, 16:32]

        def cost_block(qb, carry):
            sl = pl.ds(qb * 16, 16)
            pcx, pcy = pbT[0, sl], pbT[1, sl]
            pw, ph = pbT[2, sl], pbT[3, sl]
            px1, py1 = pcx - 0.5 * pw, pcy - 0.5 * ph
            px2, py2 = pcx + 0.5 * pw, pcy + 0.5 * ph
            area_p = (px2 - px1) * (py2 - py1)
            lsev = lse[sl]
            qidx = qb * 16 + lane
            for t in range(T):
                j = t % 16
                labt = _bcast(labA if t < 16 else labB, j)
                lg = plsc.load_gather(logitsT, [labt, qidx])
                ccls = -jnp.exp(lg - lsev)
                tcx = _bcast(tbxA if t < 16 else tbxB, j)
                tcy = _bcast(tbyA if t < 16 else tbyB, j)
                tw = _bcast(tbwA if t < 16 else tbwB, j)
                th = _bcast(tbhA if t < 16 else tbhB, j)
                cbox = (jnp.abs(pcx - tcx) + jnp.abs(pcy - tcy)
                        + jnp.abs(pw - tw) + jnp.abs(ph - th))
                tx1, ty1 = tcx - 0.5 * tw, tcy - 0.5 * th
                tx2, ty2 = tcx + 0.5 * tw, tcy + 0.5 * th
                area_t = (tx2 - tx1) * (ty2 - ty1)
                iw = jnp.maximum(jnp.minimum(px2, tx2) - jnp.maximum(px1, tx1), 0.0)
                ih = jnp.maximum(jnp.minimum(py2, ty2) - jnp.maximum(py1, ty1), 0.0)
                inter = iw * ih
                union = area_p + area_t - inter
                iou = inter / (union + 1e-8)
                hw = jnp.maximum(jnp.maximum(px2, tx2) - jnp.minimum(px1, tx1), 0.0)
                hh = jnp.maximum(jnp.maximum(py2, ty2) - jnp.minimum(py1, ty1), 0.0)
                hull = hw * hh
                giou = iou - (hull - union) / (hull + 1e-8)
                cost[t, sl] = ccls + 5.0 * cbox - 2.0 * giou
            return carry

        lax.fori_loop(0, NQB, cost_block, 0)

        used0 = tuple(
            jnp.where(qb * 16 + lane < Q, 0.0, BIG) for qb in range(NQB))

        def match_step(t, carry):
            used = carry[:NQB]
            srcA, srcB = carry[NQB], carry[NQB + 1]
            bv = _f(3e38)
            bqb = _i(0)
            for qb in range(NQB):
                v = cost[t, pl.ds(qb * 16, 16)] + used[qb]
                better = v < bv
                bv = jnp.where(better, v, bv)
                bqb = jnp.where(better, qb, bqb)
            gm = jnp.min(bv)
            qcand = jnp.where(bv == gm, bqb * 16 + lane, 100000)
            minq = jnp.min(qcand)
            mqb, mlane = minq // 16, minq % 16
            used = tuple(
                jnp.where((qb == mqb) & (lane == mlane), BIG, used[qb])
                for qb in range(NQB))
            hit = lane == (t % 16)
            srcA = jnp.where((t < 16) & hit, minq, srcA)
            srcB = jnp.where((t >= 16) & hit, minq, srcB)
            return used + (srcA, srcB)

        fin = lax.fori_loop(0, T, match_step, used0 + (_i(0), _i(0)))
        srcAB = (fin[NQB], fin[NQB + 1])

        def eos_block(qb, acc):
            sl = pl.ds(qb * 16, 16)
            valid = ((qb * 16 + lane) < Q).astype(jnp.float32)
            return acc + (lse[sl] - logitsT[C, sl]) * valid

        eos_acc = lax.fori_loop(0, NQB, eos_block, _f(0.0))
        total_eos = jnp.sum(eos_acc)

        ce_m = _f(0.0)
        bbox_m = _f(0.0)
        giou_m = _f(0.0)
        for tb in range(2):
            tmask_f = ((tb * 16 + lane) < T).astype(jnp.float32)
            sl = pl.ds(tb * 16, 16)
            src_v = srcAB[tb]
            lab_v = lab[sl]
            lse_v = plsc.load_gather(lse, [src_v])
            lg_lab = plsc.load_gather(logitsT, [lab_v, src_v])
            lg_eos = plsc.load_gather(logitsT, [_i(C), src_v])
            ce_m = ce_m + tmask_f * (
                0.1 * (lg_eos - lse_v) + (lse_v - lg_lab))
            mcx = plsc.load_gather(pbT, [_i(0), src_v])
            mcy = plsc.load_gather(pbT, [_i(1), src_v])
            mw = plsc.load_gather(pbT, [_i(2), src_v])
            mh = plsc.load_gather(pbT, [_i(3), src_v])
            tcx, tcy = tbT[0, sl], tbT[1, sl]
            tw, th = tbT[2, sl], tbT[3, sl]
            bbox_m = bbox_m + tmask_f * (
                jnp.abs(mcx - tcx) + jnp.abs(mcy - tcy)
                + jnp.abs(mw - tw) + jnp.abs(mh - th))
            mx1, my1 = mcx - 0.5 * mw, mcy - 0.5 * mh
            mx2, my2 = mcx + 0.5 * mw, mcy + 0.5 * mh
            tx1, ty1 = tcx - 0.5 * tw, tcy - 0.5 * th
            tx2, ty2 = tcx + 0.5 * tw, tcy + 0.5 * th
            area_m = (mx2 - mx1) * (my2 - my1)
            area_t = (tx2 - tx1) * (ty2 - ty1)
            iw = jnp.maximum(jnp.minimum(mx2, tx2) - jnp.maximum(mx1, tx1), 0.0)
            ih = jnp.maximum(jnp.minimum(my2, ty2) - jnp.maximum(my1, ty1), 0.0)
            inter = iw * ih
            union = area_m + area_t - inter
            iou = inter / (union + 1e-8)
            hw = jnp.maximum(jnp.maximum(mx2, tx2) - jnp.minimum(mx1, tx1), 0.0)
            hh = jnp.maximum(jnp.maximum(my2, ty2) - jnp.minimum(my1, ty1), 0.0)
            hull = hw * hh
            g = iou - (hull - union) / (hull + 1e-8)
            giou_m = giou_m + tmask_f * (1.0 - g)

        ce_part = (0.1 * total_eos + jnp.sum(ce_m)) * (1.0 / W_SUM)
        bbox_part = jnp.sum(bbox_m) * (1.0 / NUM_BOXES)
        giou_part = jnp.sum(giou_m) * (1.0 / NUM_BOXES)

        resv = jnp.where(lane == 0, ce_part,
                         jnp.where(lane == 1, bbox_part,
                                   jnp.where(lane == 2, giou_part, 0.0)))
        res[...] = resv
        pltpu.sync_copy(res, out_hbm.at[b])


@functools.partial(jax.jit, static_argnames=())
def kernel(pred_logits, pred_boxes, tgt_labels, tgt_boxes):
    logitsT = jnp.transpose(pred_logits, (0, 2, 1))
    logitsT = jnp.pad(logitsT, ((0, 0), (0, CP - NCLS), (0, QP - Q)),
                      constant_values=-1e30)
    pbT = jnp.pad(jnp.transpose(pred_boxes, (0, 2, 1)),
                  ((0, 0), (0, 0), (0, QP - Q)))
    tbT = jnp.pad(jnp.transpose(tgt_boxes, (0, 2, 1)),
                  ((0, 0), (0, 0), (0, TP - T)))
    lab = jnp.pad(tgt_labels.astype(jnp.int32), ((0, 0), (0, TP - T)))

    mesh = plsc.VectorSubcoreMesh(core_axis_name="c", subcore_axis_name="s",
                                  num_cores=2, num_subcores=16)
    out = pl.kernel(
        _sc_body,
        out_type=jax.ShapeDtypeStruct((B, 16), jnp.float32),
        mesh=mesh,
        compiler_params=pltpu.CompilerParams(use_tc_tiling_on_sc=False,
                                             needs_layout_passes=False),
        scratch_types=[
            pltpu.VMEM((CP, QP), jnp.float32),
            pltpu.VMEM((4, QP), jnp.float32),
            pltpu.VMEM((4, TP), jnp.float32),
            pltpu.VMEM((TP,), jnp.int32),
            pltpu.VMEM((QP,), jnp.float32),
            pltpu.VMEM((T, QP), jnp.float32),
            pltpu.VMEM((16,), jnp.float32),
        ],
    )(logitsT, pbT, tbT, lab)

    return (jnp.sum(out[:, 0]), jnp.sum(out[:, 1]), jnp.sum(out[:, 2]))


# speedup vs baseline: 3.4247x; 1.0459x over previous
import functools

import jax
import jax.numpy as jnp
from jax import lax
from jax.experimental import pallas as pl
from jax.experimental.pallas import tpu as pltpu
from jax.experimental.pallas import tpu_sc as plsc

B, Q, T, C = 16, 300, 20, 91
NCLS = C + 1
CP = 96
QP = 304
TP = 32
NQB = QP // 16
W_SUM = B * (0.1 * (Q - T) + 1.0 * T)
NUM_BOXES = float(B * T)
BIG = 1e30
LN2 = 0.6931471805599453


def _f(x):
    return jnp.full((16,), x, jnp.float32)


def _i(x):
    return jnp.full((16,), x, jnp.int32)


def _bcast(vec, j):
    return jnp.take_along_axis(vec, _i(j), axis=0,
                               mode=lax.GatherScatterMode.PROMISE_IN_BOUNDS)


def _log16(s):
    bits = lax.bitcast_convert_type(s, jnp.int32)
    e = ((bits >> 23) - 127).astype(jnp.float32)
    m = lax.bitcast_convert_type(
        (bits & 0x007FFFFF) | 0x3F800000, jnp.float32)
    big = m > 1.4142135
    m = jnp.where(big, 0.5 * m, m)
    e = jnp.where(big, e + 1.0, e)
    z = (m - 1.0) / (m + 1.0)
    z2 = z * z
    p = 2.0 * z * (1.0 + z2 * (1.0 / 3.0 + z2 * (0.2 + z2 * (1.0 / 7.0))))
    return e * LN2 + p


def _sc_body(logitsT_hbm, pbT_hbm, tbT_hbm, lab_hbm, out_hbm,
             logitsT, pbT, tbT, lab, lse, cost, res):
    wid = lax.axis_index("s")

    @pl.when(wid < B)
    def _run():
        b = wid
        pltpu.sync_copy(logitsT_hbm.at[b], logitsT)
        pltpu.sync_copy(pbT_hbm.at[b], pbT)
        pltpu.sync_copy(tbT_hbm.at[b], tbT)
        pltpu.sync_copy(lab_hbm.at[b], lab)

        lane = jnp.arange(16, dtype=jnp.int32)

        def lse_block(qb, carry):
            sl = pl.ds(qb * 16, 16)
            m = logitsT[0, sl]
            for c in range(1, CP):
                m = jnp.maximum(m, logitsT[c, sl])
            s = _f(0.0)
            for c in range(CP):
                s = s + jnp.exp(logitsT[c, sl] - m)
            lse[sl] = _log16(s) + m
            return carry

        lax.fori_loop(0, NQB, lse_block, 0)

        labA, labB = lab[0:16], lab[16:32]
        tbxA, tbxB = tbT[0, 0:16], tbT[0, 16:32]
        tbyA, tbyB = tbT[1, 0:16], tbT[1, 16:32]
        tbwA, tbwB = tbT[2, 0:16], tbT[2, 16:32]
        tbhA, tbhB = tbT[3, 0:16], tbT[3, 16:32]

        def cost_block(qb, carry):
            sl = pl.ds(qb * 16, 16)
            pcx, pcy = pbT[0, sl], pbT[1, sl]
            pw, ph = pbT[2, sl], pbT[3, sl]
            px1, py1 = pcx - 0.5 * pw, pcy - 0.5 * ph
            px2, py2 = pcx + 0.5 * pw, pcy + 0.5 * ph
            area_p = (px2 - px1) * (py2 - py1)
            lsev = lse[sl]
            qidx = qb * 16 + lane
            for t in range(T):
                j = t % 16
                labt = _bcast(labA if t < 16 else labB, j)
                lg = plsc.load_gather(logitsT, [labt, qidx])
                ccls = -jnp.exp(lg - lsev)
                tcx = _bcast(tbxA if t < 16 else tbxB, j)
                tcy = _bcast(tbyA if t < 16 else tbyB, j)
                tw = _bcast(tbwA if t < 16 else tbwB, j)
                th = _bcast(tbhA if t < 16 else tbhB, j)
                cbox = (jnp.abs(pcx - tcx) + jnp.abs(pcy - tcy)
                        + jnp.abs(pw - tw) + jnp.abs(ph - th))
                tx1, ty1 = tcx - 0.5 * tw, tcy - 0.5 * th
                tx2, ty2 = tcx + 0.5 * tw, tcy + 0.5 * th
                area_t = (tx2 - tx1) * (ty2 - ty1)
                iw = jnp.maximum(jnp.minimum(px2, tx2) - jnp.maximum(px1, tx1), 0.0)
                ih = jnp.maximum(jnp.minimum(py2, ty2) - jnp.maximum(py1, ty1), 0.0)
                inter = iw * ih
                union = area_p + area_t - inter
                iou = inter / (union + 1e-8)
                hw = jnp.maximum(jnp.maximum(px2, tx2) - jnp.minimum(px1, tx1), 0.0)
                hh = jnp.maximum(jnp.maximum(py2, ty2) - jnp.minimum(py1, ty1), 0.0)
                hull = hw * hh
                giou = iou - (hull - union) / (hull + 1e-8)
                cost[t, sl] = ccls + 5.0 * cbox - 2.0 * giou
            return carry

        lax.fori_loop(0, NQB, cost_block, 0)

        used0 = tuple(
            jnp.where(qb * 16 + lane < Q, 0.0, BIG) for qb in range(NQB))

        def match_step(t, carry):
            used = carry[:NQB]
            srcA, srcB = carry[NQB], carry[NQB + 1]
            bv = _f(3e38)
            bqb = _i(0)
            for qb in range(NQB):
                v = cost[t, pl.ds(qb * 16, 16)] + used[qb]
                better = v < bv
                bv = jnp.where(better, v, bv)
                bqb = jnp.where(better, qb, bqb)
            gm = jnp.min(bv)
            qcand = jnp.where(bv == gm, bqb * 16 + lane, 100000)
            minq = jnp.min(qcand)
            mqb, mlane = minq // 16, minq % 16
            used = tuple(
                jnp.where((qb == mqb) & (lane == mlane), BIG, used[qb])
                for qb in range(NQB))
            hit = lane == (t % 16)
            srcA = jnp.where((t < 16) & hit, minq, srcA)
            srcB = jnp.where((t >= 16) & hit, minq, srcB)
            return used + (srcA, srcB)

        fin = lax.fori_loop(0, T, match_step, used0 + (_i(0), _i(0)))
        srcAB = (fin[NQB], fin[NQB + 1])

        def eos_block(qb, acc):
            sl = pl.ds(qb * 16, 16)
            valid = ((qb * 16 + lane) < Q).astype(jnp.float32)
            return acc + (lse[sl] - logitsT[C, sl]) * valid

        eos_acc = lax.fori_loop(0, NQB, eos_block, _f(0.0))
        total_eos = jnp.sum(eos_acc)

        ce_m = _f(0.0)
        bbox_m = _f(0.0)
        giou_m = _f(0.0)
        for tb in range(2):
            tmask_f = ((tb * 16 + lane) < T).astype(jnp.float32)
            sl = pl.ds(tb * 16, 16)
            src_v = srcAB[tb]
            lab_v = lab[sl]
            lse_v = plsc.load_gather(lse, [src_v])
            lg_lab = plsc.load_gather(logitsT, [lab_v, src_v])
            lg_eos = plsc.load_gather(logitsT, [_i(C), src_v])
            ce_m = ce_m + tmask_f * (
                0.1 * (lg_eos - lse_v) + (lse_v - lg_lab))
            mcx = plsc.load_gather(pbT, [_i(0), src_v])
            mcy = plsc.load_gather(pbT, [_i(1), src_v])
            mw = plsc.load_gather(pbT, [_i(2), src_v])
            mh = plsc.load_gather(pbT, [_i(3), src_v])
            tcx, tcy = tbT[0, sl], tbT[1, sl]
            tw, th = tbT[2, sl], tbT[3, sl]
            bbox_m = bbox_m + tmask_f * (
                jnp.abs(mcx - tcx) + jnp.abs(mcy - tcy)
                + jnp.abs(mw - tw) + jnp.abs(mh - th))
            mx1, my1 = mcx - 0.5 * mw, mcy - 0.5 * mh
            mx2, my2 = mcx + 0.5 * mw, mcy + 0.5 * mh
            tx1, ty1 = tcx - 0.5 * tw, tcy - 0.5 * th
            tx2, ty2 = tcx + 0.5 * tw, tcy + 0.5 * th
            area_m = (mx2 - mx1) * (my2 - my1)
            area_t = (tx2 - tx1) * (ty2 - ty1)
            iw = jnp.maximum(jnp.minimum(mx2, tx2) - jnp.maximum(mx1, tx1), 0.0)
            ih = jnp.maximum(jnp.minimum(my2, ty2) - jnp.maximum(my1, ty1), 0.0)
            inter = iw * ih
            union = area_m + area_t - inter
            iou = inter / (union + 1e-8)
            hw = jnp.maximum(jnp.maximum(mx2, tx2) - jnp.minimum(mx1, tx1), 0.0)
            hh = jnp.maximum(jnp.maximum(my2, ty2) - jnp.minimum(my1, ty1), 0.0)
            hull = hw * hh
            g = iou - (hull - union) / (hull + 1e-8)
            giou_m = giou_m + tmask_f * (1.0 - g)

        ce_part = (0.1 * total_eos + jnp.sum(ce_m)) * (1.0 / W_SUM)
        bbox_part = jnp.sum(bbox_m) * (1.0 / NUM_BOXES)
        giou_part = jnp.sum(giou_m) * (1.0 / NUM_BOXES)

        resv = jnp.where(lane == 0, ce_part,
                         jnp.where(lane == 1, bbox_part,
                                   jnp.where(lane == 2, giou_part, 0.0)))
        res[...] = resv
        pltpu.sync_copy(res, out_hbm.at[b])


@functools.partial(jax.jit, static_argnames=())
def kernel(pred_logits, pred_boxes, tgt_labels, tgt_boxes):
    logitsT = jnp.transpose(pred_logits, (0, 2, 1))
    logitsT = jnp.pad(logitsT, ((0, 0), (0, CP - NCLS), (0, QP - Q)),
                      constant_values=-1e30)
    pbT = jnp.pad(jnp.transpose(pred_boxes, (0, 2, 1)),
                  ((0, 0), (0, 0), (0, QP - Q)))
    tbT = jnp.pad(jnp.transpose(tgt_boxes, (0, 2, 1)),
                  ((0, 0), (0, 0), (0, TP - T)))
    lab = jnp.pad(tgt_labels.astype(jnp.int32), ((0, 0), (0, TP - T)))

    mesh = plsc.VectorSubcoreMesh(core_axis_name="c", subcore_axis_name="s",
                                  num_cores=1, num_subcores=16)
    out = pl.kernel(
        _sc_body,
        out_type=jax.ShapeDtypeStruct((B, 16), jnp.float32),
        mesh=mesh,
        compiler_params=pltpu.CompilerParams(use_tc_tiling_on_sc=False,
                                             needs_layout_passes=False),
        scratch_types=[
            pltpu.VMEM((CP, QP), jnp.float32),
            pltpu.VMEM((4, QP), jnp.float32),
            pltpu.VMEM((4, TP), jnp.float32),
            pltpu.VMEM((TP,), jnp.int32),
            pltpu.VMEM((QP,), jnp.float32),
            pltpu.VMEM((T, QP), jnp.float32),
            pltpu.VMEM((16,), jnp.float32),
        ],
    )(logitsT, pbT, tbT, lab)

    return (jnp.sum(out[:, 0]), jnp.sum(out[:, 1]), jnp.sum(out[:, 2]))
